# baseline (speedup 1.0000x reference)
import jax
import jax.numpy as jnp
from jax.experimental import pallas as pl
from jax.experimental.pallas import tpu as pltpu

_NF = 2


def _fused_kernel(x_ref, w_ref, m_ref, b_ref, o_ref, wbf_ref):
    i = pl.program_id(0)
    chunk = w_ref.shape[0]

    @pl.when(i < _NF)
    def _():
        wbf_ref[pl.ds(i * chunk, chunk), :] = (
            w_ref[...] * m_ref[...]).astype(jnp.bfloat16)

    @pl.when(i >= _NF)
    def _():
        xb = x_ref[...].astype(jnp.bfloat16)
        half = o_ref.shape[1] // 2
        y0 = jnp.dot(xb, wbf_ref[:, :half],
                     preferred_element_type=jnp.float32)
        o_ref[:, :half] = jnp.maximum(y0 + b_ref[:, :half], 0.0
                                      ).astype(o_ref.dtype)
        y1 = jnp.dot(xb, wbf_ref[:, half:],
                     preferred_element_type=jnp.float32)
        o_ref[:, half:] = jnp.maximum(y1 + b_ref[:, half:], 0.0
                                      ).astype(o_ref.dtype)


def kernel(x, weight, mask, bias):
    B, n_in = x.shape
    n_out = weight.shape[1]
    bias2d = bias.astype(jnp.float32).reshape(1, n_out)

    tm = 512
    chunk = n_in // _NF
    ns = B // tm

    out = pl.pallas_call(
        _fused_kernel,
        out_shape=jax.ShapeDtypeStruct((B, n_out), x.dtype),
        grid=(_NF + ns,),
        in_specs=[
            pl.BlockSpec((tm, n_in), lambda i: (jnp.maximum(i - _NF, 0), 0)),
            pl.BlockSpec((chunk, n_out), lambda i: (jnp.minimum(i, _NF - 1), 0)),
            pl.BlockSpec((chunk, n_out), lambda i: (jnp.minimum(i, _NF - 1), 0)),
            pl.BlockSpec((1, n_out), lambda i: (0, 0)),
        ],
        out_specs=pl.BlockSpec((tm, n_out), lambda i: (jnp.maximum(i - _NF, 0), 0)),
        scratch_shapes=[pltpu.VMEM((n_in, n_out), jnp.bfloat16)],
        compiler_params=pltpu.CompilerParams(
            dimension_semantics=("arbitrary",)),
    )(x, weight, mask, bias2d)
    return out


# speedup vs baseline: 1.0031x; 1.0031x over previous
import jax
import jax.numpy as jnp
from jax.experimental import pallas as pl
from jax.experimental.pallas import tpu as pltpu

_NF = 4


def _fused_kernel(x_ref, w_ref, m_ref, b_ref, o_ref, wbf_ref):
    i = pl.program_id(0)
    chunk = w_ref.shape[0]

    @pl.when(i < _NF)
    def _():
        wbf_ref[pl.ds(i * chunk, chunk), :] = (
            w_ref[...] * m_ref[...]).astype(jnp.bfloat16)

    @pl.when(i >= _NF)
    def _():
        xb = x_ref[...].astype(jnp.bfloat16)
        quarter = o_ref.shape[1] // 4
        for q in range(4):
            sl = pl.ds(q * quarter, quarter)
            y = jnp.dot(xb, wbf_ref[:, sl],
                        preferred_element_type=jnp.float32)
            o_ref[:, sl] = jnp.maximum(y + b_ref[:, sl], 0.0
                                       ).astype(o_ref.dtype)


def kernel(x, weight, mask, bias):
    B, n_in = x.shape
    n_out = weight.shape[1]
    bias2d = bias.astype(jnp.float32).reshape(1, n_out)

    tm = 512
    chunk = n_in // _NF
    ns = B // tm

    out = pl.pallas_call(
        _fused_kernel,
        out_shape=jax.ShapeDtypeStruct((B, n_out), x.dtype),
        grid=(_NF + ns,),
        in_specs=[
            pl.BlockSpec((tm, n_in), lambda i: (jnp.maximum(i - _NF, 0), 0)),
            pl.BlockSpec((chunk, n_out), lambda i: (jnp.minimum(i, _NF - 1), 0)),
            pl.BlockSpec((chunk, n_out), lambda i: (jnp.minimum(i, _NF - 1), 0)),
            pl.BlockSpec((1, n_out), lambda i: (0, 0)),
        ],
        out_specs=pl.BlockSpec((tm, n_out), lambda i: (jnp.maximum(i - _NF, 0), 0)),
        scratch_shapes=[pltpu.VMEM((n_in, n_out), jnp.bfloat16)],
        compiler_params=pltpu.CompilerParams(
            dimension_semantics=("arbitrary",)),
    )(x, weight, mask, bias2d)
    return out


# speedup vs baseline: 1.0246x; 1.0214x over previous
import jax
import jax.numpy as jnp
from jax.experimental import pallas as pl
from jax.experimental.pallas import tpu as pltpu

_NF = 4


def _fused_kernel(x_ref, w_ref, m_ref, b_ref, o_ref, wbf_ref, acc_ref):
    i = pl.program_id(0)
    chunk = w_ref.shape[0]

    @pl.when(i < _NF)
    def _():
        wc = (w_ref[...] * m_ref[...]).astype(jnp.bfloat16)
        wbf_ref[pl.ds(i * chunk, chunk), :] = wc
        xb = x_ref[:, pl.ds(i * chunk, chunk)].astype(jnp.bfloat16)
        y = jnp.dot(xb, wc, preferred_element_type=jnp.float32)

        @pl.when(i == 0)
        def _():
            acc_ref[...] = y

        @pl.when(i > 0)
        def _():
            acc_ref[...] += y

        @pl.when(i == _NF - 1)
        def _():
            o_ref[...] = jnp.maximum(acc_ref[...] + b_ref[...], 0.0
                                     ).astype(o_ref.dtype)

    @pl.when(i >= _NF)
    def _():
        xb = x_ref[...].astype(jnp.bfloat16)
        quarter = o_ref.shape[1] // 4
        for q in range(4):
            sl = pl.ds(q * quarter, quarter)
            y = jnp.dot(xb, wbf_ref[:, sl],
                        preferred_element_type=jnp.float32)
            o_ref[:, sl] = jnp.maximum(y + b_ref[:, sl], 0.0
                                       ).astype(o_ref.dtype)


def kernel(x, weight, mask, bias):
    B, n_in = x.shape
    n_out = weight.shape[1]
    bias2d = bias.astype(jnp.float32).reshape(1, n_out)

    tm = 512
    chunk = n_in // _NF
    ns = B // tm

    out = pl.pallas_call(
        _fused_kernel,
        out_shape=jax.ShapeDtypeStruct((B, n_out), x.dtype),
        grid=(_NF + ns - 1,),
        in_specs=[
            pl.BlockSpec((tm, n_in), lambda i: (jnp.maximum(i - _NF + 1, 0), 0)),
            pl.BlockSpec((chunk, n_out), lambda i: (jnp.minimum(i, _NF - 1), 0)),
            pl.BlockSpec((chunk, n_out), lambda i: (jnp.minimum(i, _NF - 1), 0)),
            pl.BlockSpec((1, n_out), lambda i: (0, 0)),
        ],
        out_specs=pl.BlockSpec(
            (tm, n_out), lambda i: (jnp.maximum(i - _NF + 1, 0), 0)),
        scratch_shapes=[pltpu.VMEM((n_in, n_out), jnp.bfloat16),
                        pltpu.VMEM((tm, n_out), jnp.float32)],
        compiler_params=pltpu.CompilerParams(
            dimension_semantics=("arbitrary",)),
    )(x, weight, mask, bias2d)
    return out
